# baseline (speedup 1.0000x reference)
import functools

import jax
import jax.numpy as jnp
from jax import lax
from jax.experimental import pallas as pl
from jax.experimental.pallas import tpu as pltpu
from jax.experimental.pallas import tpu_sc as plsc

NC = 2
NS = 16
LANES = 16
CHUNK = 128
KJ = 4
KI = 32

D = 48
HALF = 24
CPAD = 32


def _ceil_to(a, m):
    return (a + m - 1) // m * m


def _sc_mesh():
    return plsc.VectorSubcoreMesh(
        core_axis_name="c", subcore_axis_name="s",
        num_cores=NC, num_subcores=NS,
    )


def _make_deg_kernel(n_t, n_chunk_rows):
    kid = KI // 2
    rows_per_tile = n_t // NS
    rows_per_cr = n_chunk_rows // (NC * NS)
    n_outer = rows_per_cr // kid
    assert rows_per_cr % kid == 0
    nz_full = rows_per_tile // CHUNK

    @functools.partial(
        pl.kernel,
        out_type=jax.ShapeDtypeStruct((NC * n_t,), jnp.float32),
        mesh=_sc_mesh(),
        scratch_types=[
            pltpu.VMEM((kid, CHUNK), jnp.int32),
            pltpu.VMEM((CHUNK,), jnp.float32),
            pltpu.VMEM((CHUNK,), jnp.float32),
            pltpu.VMEM_SHARED((n_t,), jnp.float32),
        ],
        compiler_params=pltpu.CompilerParams(use_tc_tiling_on_sc=False),
    )
    def deg_kernel(dst_hbm, deg_out, idx_buf, ones_buf, zero_buf, deg_sp):
        c = lax.axis_index("c")
        s = lax.axis_index("s")
        one = jnp.full((LANES,), 1.0, jnp.float32)
        zero = jnp.zeros((LANES,), jnp.float32)
        for i in range(CHUNK // LANES):
            ones_buf[pl.ds(i * LANES, LANES)] = one
            zero_buf[pl.ds(i * LANES, LANES)] = zero
        base = s * rows_per_tile

        def zbody(o, carry):
            pltpu.sync_copy(zero_buf, deg_sp.at[pl.ds(base + o * CHUNK, CHUNK)])
            return carry

        lax.fori_loop(0, nz_full, zbody, 0)
        if rows_per_tile % CHUNK != 0:
            pltpu.sync_copy(
                zero_buf, deg_sp.at[pl.ds(base + rows_per_tile - CHUNK, CHUNK)]
            )
        plsc.subcore_barrier()

        wid = s * NC + c
        crow0 = wid * rows_per_cr

        def obody(o, carry):
            pltpu.sync_copy(dst_hbm.at[pl.ds(crow0 + o * kid, kid)], idx_buf)
            for j in range(kid):
                pltpu.sync_copy(ones_buf, deg_sp.at[idx_buf.at[j]], add=True)
            return carry

        lax.fori_loop(0, n_outer, obody, 0)
        plsc.subcore_barrier()
        pltpu.sync_copy(
            deg_sp.at[pl.ds(base, rows_per_tile)],
            deg_out.at[pl.ds(c * n_t + base, rows_per_tile)],
        )

    return deg_kernel


def _make_scatter_kernel(n_t, n_chunk_rows):
    rows_per_tile = n_t // NS
    rows_per_cr = n_chunk_rows // NS
    n_outer = rows_per_cr // KI
    assert rows_per_cr % KI == 0
    assert KI % KJ == 0
    nz_full = rows_per_tile // CHUNK

    @functools.partial(
        pl.kernel,
        out_type=jax.ShapeDtypeStruct((NC * n_t, CPAD), jnp.float32),
        mesh=_sc_mesh(),
        scratch_types=[
            pltpu.VMEM((KI, CHUNK), jnp.int32),
            pltpu.VMEM((KI, CHUNK), jnp.int32),
            pltpu.VMEM((KJ, CHUNK, CPAD), jnp.float32),
            pltpu.VMEM_SHARED((n_t, CPAD), jnp.float32),
            pltpu.SemaphoreType.DMA,
            pltpu.SemaphoreType.DMA,
        ],
        compiler_params=pltpu.CompilerParams(use_tc_tiling_on_sc=False),
    )
    def scatter_kernel(src2_hbm, dst2_hbm, y2_hbm, acc_out,
                       sidx, didx, rows, acc_sp, sem0, sem1):
        c = lax.axis_index("c")
        s = lax.axis_index("s")

        z = jnp.zeros((LANES,), jnp.float32)

        def zr(r, carry):
            for k in range(CPAD // LANES):
                rows[0, r, pl.ds(k * LANES, LANES)] = z
            return carry

        lax.fori_loop(0, CHUNK, zr, 0)
        base = s * rows_per_tile

        def zbody(o, carry):
            pltpu.sync_copy(rows.at[0], acc_sp.at[pl.ds(base + o * CHUNK, CHUNK)])
            return carry

        lax.fori_loop(0, nz_full, zbody, 0)
        if rows_per_tile % CHUNK != 0:
            pltpu.sync_copy(
                rows.at[0], acc_sp.at[pl.ds(base + rows_per_tile - CHUNK, CHUNK)]
            )
        plsc.subcore_barrier()

        crow0 = s * rows_per_cr

        kjh = KJ // 2

        def obody(o, carry):
            pltpu.sync_copy(src2_hbm.at[c, pl.ds(crow0 + o * KI, KI)], sidx)
            pltpu.sync_copy(dst2_hbm.at[pl.ds(crow0 + o * KI, KI)], didx)
            for t in range(KI // KJ):
                b = t * KJ
                g0 = [
                    pltpu.async_copy(y2_hbm.at[sidx.at[b + j]], rows.at[j], sem0)
                    for j in range(kjh)
                ]
                g1 = [
                    pltpu.async_copy(y2_hbm.at[sidx.at[b + j]], rows.at[j], sem1)
                    for j in range(kjh, KJ)
                ]
                for d in g0:
                    d.wait()
                for j in range(kjh):
                    pltpu.sync_copy(rows.at[j], acc_sp.at[didx.at[b + j]], add=True)
                for d in g1:
                    d.wait()
                for j in range(kjh, KJ):
                    pltpu.sync_copy(rows.at[j], acc_sp.at[didx.at[b + j]], add=True)
            return carry

        lax.fori_loop(0, n_outer, obody, 0)
        plsc.subcore_barrier()
        pltpu.sync_copy(
            acc_sp.at[pl.ds(base, rows_per_tile)],
            acc_out.at[pl.ds(c * n_t + base, rows_per_tile)],
        )

    return scatter_kernel


def _mm_body(x_ref, h_ref, q_ref, deg_ref, wt_ref, y2_ref):
    inp = jnp.concatenate([x_ref[...], h_ref[...], q_ref[...]], axis=1)
    xw = jnp.dot(inp, wt_ref[...], preferred_element_type=jnp.float32)
    deg = deg_ref[:, 0] + deg_ref[:, 1] + 1.0
    dis = lax.rsqrt(deg)
    y = xw * dis[:, None]
    if CPAD > HALF:
        zpad = jnp.zeros((y.shape[0], CPAD - HALF), jnp.float32)
        y2_ref[0] = jnp.concatenate([y[:, :HALF], zpad], axis=1)
        y2_ref[1] = jnp.concatenate([y[:, HALF:], zpad], axis=1)
    else:
        y2_ref[0] = y[:, :HALF]
        y2_ref[1] = y[:, HALF:]


def _fin_body(acc_ref, y2_ref, deg_ref, b_ref, out_ref):
    deg = deg_ref[:, 0] + deg_ref[:, 1] + 1.0
    dis = lax.rsqrt(deg)
    left = acc_ref[0, :, :HALF] + y2_ref[0, :, :HALF]
    right = acc_ref[1, :, :HALF] + y2_ref[1, :, :HALF]
    out_ref[...] = (
        jnp.concatenate([left, right], axis=1) * dis[:, None] + b_ref[0]
    )


def kernel(h, e, x, q, mask, Ws, bs):
    n = h.shape[0]
    n_edges = e.shape[1]
    n_t = _ceil_to(n + 1, NS * CHUNK)
    e_pad = _ceil_to(n_edges, NS * CHUNK * KI)
    n_chunk_rows = e_pad // CHUNK

    src = e[0].astype(jnp.int32)
    dst = e[1].astype(jnp.int32)
    pad = e_pad - n_edges
    padv = jnp.full((pad,), n, jnp.int32)
    srcp = jnp.concatenate([src, padv])
    dstp = jnp.concatenate([dst, padv])
    dst2 = dstp.reshape(n_chunk_rows, CHUNK)
    src2 = jnp.stack(
        [srcp.reshape(n_chunk_rows, CHUNK),
         (srcp + n_t).reshape(n_chunk_rows, CHUNK)]
    )

    rpad = n_t - n
    xp = jnp.pad(x, ((0, rpad), (0, 0)))
    hp = jnp.pad(h, ((0, rpad), (0, 0)))
    qp = jnp.pad(q, ((0, rpad), (0, 0)))
    wt = Ws[-1].T
    b2 = bs[-1].reshape(1, D)

    deg = _make_deg_kernel(n_t, n_chunk_rows)(dst2)
    deg = deg.reshape(NC, n_t).T

    bn = n_t // 16
    grid = n_t // bn
    y2 = pl.pallas_call(
        _mm_body,
        grid=(grid,),
        in_specs=[
            pl.BlockSpec((bn, 16), lambda i: (i, 0)),
            pl.BlockSpec((bn, 16), lambda i: (i, 0)),
            pl.BlockSpec((bn, 16), lambda i: (i, 0)),
            pl.BlockSpec((bn, NC), lambda i: (i, 0)),
            pl.BlockSpec((D, D), lambda i: (0, 0)),
        ],
        out_specs=pl.BlockSpec((NC, bn, CPAD), lambda i: (0, i, 0)),
        out_shape=jax.ShapeDtypeStruct((NC, n_t, CPAD), jnp.float32),
    )(xp, hp, qp, deg, wt)

    acc = _make_scatter_kernel(n_t, n_chunk_rows)(
        src2, dst2, y2.reshape(NC * n_t, CPAD)
    ).reshape(NC, n_t, CPAD)

    out = pl.pallas_call(
        _fin_body,
        grid=(grid,),
        in_specs=[
            pl.BlockSpec((NC, bn, CPAD), lambda i: (0, i, 0)),
            pl.BlockSpec((NC, bn, CPAD), lambda i: (0, i, 0)),
            pl.BlockSpec((bn, NC), lambda i: (i, 0)),
            pl.BlockSpec((1, D), lambda i: (0, 0)),
        ],
        out_specs=pl.BlockSpec((bn, D), lambda i: (i, 0)),
        out_shape=jax.ShapeDtypeStruct((n_t, D), jnp.float32),
    )(acc, y2, deg, b2)

    return out[:n]


# speedup vs baseline: 1.4418x; 1.4418x over previous
import functools

import jax
import jax.numpy as jnp
from jax import lax
from jax.experimental import pallas as pl
from jax.experimental.pallas import tpu as pltpu
from jax.experimental.pallas import tpu_sc as plsc

NC = 2
NS = 16
LANES = 16
CHUNK = 128
KJ = 4
KI = 32

D = 48
HALF = 24
CPAD = 32


def _ceil_to(a, m):
    return (a + m - 1) // m * m


def _sc_mesh():
    return plsc.VectorSubcoreMesh(
        core_axis_name="c", subcore_axis_name="s",
        num_cores=NC, num_subcores=NS,
    )


def _make_deg_kernel(n_t, n_chunk_rows):
    kid = KI // 2
    rows_per_tile = n_t // NS
    rows_per_cr = n_chunk_rows // (NC * NS)
    n_outer = rows_per_cr // kid
    assert rows_per_cr % kid == 0
    nz_full = rows_per_tile // CHUNK

    @functools.partial(
        pl.kernel,
        out_type=jax.ShapeDtypeStruct((NC * n_t,), jnp.float32),
        mesh=_sc_mesh(),
        scratch_types=[
            pltpu.VMEM((kid, CHUNK), jnp.int32),
            pltpu.VMEM((CHUNK,), jnp.float32),
            pltpu.VMEM((CHUNK,), jnp.float32),
            pltpu.VMEM_SHARED((n_t,), jnp.float32),
        ],
        compiler_params=pltpu.CompilerParams(use_tc_tiling_on_sc=False),
    )
    def deg_kernel(dst_hbm, deg_out, idx_buf, ones_buf, zero_buf, deg_sp):
        c = lax.axis_index("c")
        s = lax.axis_index("s")
        one = jnp.full((LANES,), 1.0, jnp.float32)
        zero = jnp.zeros((LANES,), jnp.float32)
        for i in range(CHUNK // LANES):
            ones_buf[pl.ds(i * LANES, LANES)] = one
            zero_buf[pl.ds(i * LANES, LANES)] = zero
        base = s * rows_per_tile

        def zbody(o, carry):
            pltpu.sync_copy(zero_buf, deg_sp.at[pl.ds(base + o * CHUNK, CHUNK)])
            return carry

        lax.fori_loop(0, nz_full, zbody, 0)
        if rows_per_tile % CHUNK != 0:
            pltpu.sync_copy(
                zero_buf, deg_sp.at[pl.ds(base + rows_per_tile - CHUNK, CHUNK)]
            )
        plsc.subcore_barrier()

        wid = s * NC + c
        crow0 = wid * rows_per_cr

        def obody(o, carry):
            pltpu.sync_copy(dst_hbm.at[pl.ds(crow0 + o * kid, kid)], idx_buf)
            for j in range(kid):
                pltpu.sync_copy(ones_buf, deg_sp.at[idx_buf.at[j]], add=True)
            return carry

        lax.fori_loop(0, n_outer, obody, 0)
        plsc.subcore_barrier()
        pltpu.sync_copy(
            deg_sp.at[pl.ds(base, rows_per_tile)],
            deg_out.at[pl.ds(c * n_t + base, rows_per_tile)],
        )

    return deg_kernel


def _make_scatter_kernel(n_t, n_chunk_rows):
    rows_per_tile = n_t // NS
    rows_per_cr = n_chunk_rows // NS
    n_outer = rows_per_cr // KI
    assert rows_per_cr % KI == 0
    assert KI % KJ == 0
    nz_full = rows_per_tile // CHUNK

    @functools.partial(
        pl.kernel,
        out_type=jax.ShapeDtypeStruct((NC * n_t, CPAD), jnp.float32),
        mesh=_sc_mesh(),
        scratch_types=[
            pltpu.VMEM((KI, CHUNK), jnp.int32),
            pltpu.VMEM((KI, CHUNK), jnp.int32),
            pltpu.VMEM((KJ, CHUNK, CPAD), jnp.float32),
            pltpu.VMEM_SHARED((n_t, CPAD), jnp.float32),
            pltpu.SemaphoreType.DMA,
            pltpu.SemaphoreType.DMA,
        ],
        compiler_params=pltpu.CompilerParams(use_tc_tiling_on_sc=False),
    )
    def scatter_kernel(src2_hbm, dst2_hbm, y2_hbm, acc_out,
                       sidx, didx, rows, acc_sp, sem0, sem1):
        c = lax.axis_index("c")
        s = lax.axis_index("s")

        z = jnp.zeros((LANES,), jnp.float32)

        def zr(r, carry):
            for k in range(CPAD // LANES):
                rows[0, r, pl.ds(k * LANES, LANES)] = z
            return carry

        lax.fori_loop(0, CHUNK, zr, 0)
        base = s * rows_per_tile

        def zbody(o, carry):
            pltpu.sync_copy(rows.at[0], acc_sp.at[pl.ds(base + o * CHUNK, CHUNK)])
            return carry

        lax.fori_loop(0, nz_full, zbody, 0)
        if rows_per_tile % CHUNK != 0:
            pltpu.sync_copy(
                rows.at[0], acc_sp.at[pl.ds(base + rows_per_tile - CHUNK, CHUNK)]
            )
        plsc.subcore_barrier()

        crow0 = s * rows_per_cr

        kjh = KJ // 2

        def obody(o, carry):
            pltpu.sync_copy(src2_hbm.at[c, pl.ds(crow0 + o * KI, KI)], sidx)
            pltpu.sync_copy(dst2_hbm.at[pl.ds(crow0 + o * KI, KI)], didx)
            for t in range(KI // KJ):
                b = t * KJ
                g0 = [
                    pltpu.async_copy(y2_hbm.at[sidx.at[b + j]], rows.at[j], sem0)
                    for j in range(kjh)
                ]
                g1 = [
                    pltpu.async_copy(y2_hbm.at[sidx.at[b + j]], rows.at[j], sem1)
                    for j in range(kjh, KJ)
                ]
                for d in g0:
                    d.wait()
                for j in range(kjh):
                    pltpu.sync_copy(rows.at[j], acc_sp.at[didx.at[b + j]], add=True)
                for d in g1:
                    d.wait()
                for j in range(kjh, KJ):
                    pltpu.sync_copy(rows.at[j], acc_sp.at[didx.at[b + j]], add=True)
            return carry

        lax.fori_loop(0, n_outer, obody, 0)
        plsc.subcore_barrier()
        pltpu.sync_copy(
            acc_sp.at[pl.ds(base, rows_per_tile)],
            acc_out.at[pl.ds(c * n_t + base, rows_per_tile)],
        )

    return scatter_kernel


def _mm_body(x_ref, h_ref, q_ref, deg_ref, wt_ref, y2_ref):
    inp = jnp.concatenate([x_ref[...], h_ref[...], q_ref[...]], axis=1)
    xw = jnp.dot(inp, wt_ref[...], preferred_element_type=jnp.float32)
    deg = deg_ref[:, 0] + deg_ref[:, 1] + 1.0
    dis = lax.rsqrt(deg)
    y = xw * dis[:, None]
    if CPAD > HALF:
        zpad = jnp.zeros((y.shape[0], CPAD - HALF), jnp.float32)
        y2_ref[0] = jnp.concatenate([y[:, :HALF], zpad], axis=1)
        y2_ref[1] = jnp.concatenate([y[:, HALF:], zpad], axis=1)
    else:
        y2_ref[0] = y[:, :HALF]
        y2_ref[1] = y[:, HALF:]


def _fin_body(acc_ref, y2_ref, deg_ref, b_ref, out_ref):
    deg = deg_ref[:, 0] + deg_ref[:, 1] + 1.0
    dis = lax.rsqrt(deg)
    left = acc_ref[0, :, :HALF] + y2_ref[0, :, :HALF]
    right = acc_ref[1, :, :HALF] + y2_ref[1, :, :HALF]
    out_ref[...] = (
        jnp.concatenate([left, right], axis=1) * dis[:, None] + b_ref[0]
    )


def kernel(h, e, x, q, mask, Ws, bs):
    n = h.shape[0]
    n_edges = e.shape[1]
    n_t = _ceil_to(n + 1, NS * CHUNK)
    e_pad = _ceil_to(n_edges, NS * CHUNK * KI)
    n_chunk_rows = e_pad // CHUNK

    src = e[0].astype(jnp.int32)
    dst = e[1].astype(jnp.int32)
    pad = e_pad - n_edges
    padv = n + jnp.arange(pad, dtype=jnp.int32) % (n_t - n)
    srcp = jnp.concatenate([src, padv])
    dstp = jnp.concatenate([dst, padv])
    dst2 = dstp.reshape(n_chunk_rows, CHUNK)
    src2 = jnp.stack(
        [srcp.reshape(n_chunk_rows, CHUNK),
         (srcp + n_t).reshape(n_chunk_rows, CHUNK)]
    )

    rpad = n_t - n
    xp = jnp.pad(x, ((0, rpad), (0, 0)))
    hp = jnp.pad(h, ((0, rpad), (0, 0)))
    qp = jnp.pad(q, ((0, rpad), (0, 0)))
    wt = Ws[-1].T
    b2 = bs[-1].reshape(1, D)

    deg = _make_deg_kernel(n_t, n_chunk_rows)(dst2)
    deg = deg.reshape(NC, n_t).T

    bn = n_t // 16
    grid = n_t // bn
    y2 = pl.pallas_call(
        _mm_body,
        grid=(grid,),
        in_specs=[
            pl.BlockSpec((bn, 16), lambda i: (i, 0)),
            pl.BlockSpec((bn, 16), lambda i: (i, 0)),
            pl.BlockSpec((bn, 16), lambda i: (i, 0)),
            pl.BlockSpec((bn, NC), lambda i: (i, 0)),
            pl.BlockSpec((D, D), lambda i: (0, 0)),
        ],
        out_specs=pl.BlockSpec((NC, bn, CPAD), lambda i: (0, i, 0)),
        out_shape=jax.ShapeDtypeStruct((NC, n_t, CPAD), jnp.float32),
    )(xp, hp, qp, deg, wt)

    acc = _make_scatter_kernel(n_t, n_chunk_rows)(
        src2, dst2, y2.reshape(NC * n_t, CPAD)
    ).reshape(NC, n_t, CPAD)

    out = pl.pallas_call(
        _fin_body,
        grid=(grid,),
        in_specs=[
            pl.BlockSpec((NC, bn, CPAD), lambda i: (0, i, 0)),
            pl.BlockSpec((NC, bn, CPAD), lambda i: (0, i, 0)),
            pl.BlockSpec((bn, NC), lambda i: (i, 0)),
            pl.BlockSpec((1, D), lambda i: (0, 0)),
        ],
        out_specs=pl.BlockSpec((bn, D), lambda i: (i, 0)),
        out_shape=jax.ShapeDtypeStruct((n_t, D), jnp.float32),
    )(acc, y2, deg, b2)

    return out[:n]
